# baseline (speedup 1.0000x reference)
import functools

import jax
import jax.numpy as jnp
from jax import lax
from jax.experimental import pallas as pl
from jax.experimental.pallas import tpu as pltpu
from jax.experimental.pallas import tpu_sc as plsc

VOCAB = 100000
D = 128
S = 200
B = 4096

NC = 2
NS = 16
L = 16
BT = B // (NC * NS)
SC_B = B // NC


def _mean_embed(sentence, table):
    mesh = plsc.VectorSubcoreMesh(core_axis_name="c", subcore_axis_name="s")

    @functools.partial(
        pl.kernel,
        mesh=mesh,
        out_type=jax.ShapeDtypeStruct((B, D), jnp.float32),
        scratch_types=[
            pltpu.VMEM((S, BT), jnp.int32),
            pltpu.VMEM((5, BT, D), jnp.float32),
            pltpu.VMEM((BT,), jnp.int32),
            pltpu.VMEM_SHARED((SC_B, D), jnp.float32),
            [pltpu.SemaphoreType.DMA] * 5,
            [pltpu.SemaphoreType.DMA] * 5,
        ],
    )
    def k(sent_hbm, table_hbm, out_hbm, idx_v, rows_v, dst_v,
          accum_sh, gsems, ssems):
        cid = lax.axis_index("c")
        sid = lax.axis_index("s")
        tid = cid * NS + sid
        gbase = tid * BT
        lbase = sid * BT

        pltpu.sync_copy(sent_hbm.at[:, pl.ds(gbase, BT)], idx_v)

        for j in range(BT // L):
            dst_v[pl.ds(j * L, L)] = (
                jnp.full((L,), lbase + j * L, jnp.int32)
                + lax.iota(jnp.int32, L)
            )

        NB = 5

        def wait_gather(b):
            pltpu.make_async_copy(
                table_hbm.at[idx_v.at[0]], rows_v.at[b], gsems[b]
            ).wait()

        def wait_scatter(b):
            pltpu.make_async_copy(
                rows_v.at[b], accum_sh.at[dst_v], ssems[b]
            ).wait()

        for b in range(NB):
            pltpu.async_copy(table_hbm.at[idx_v.at[b]], rows_v.at[b], gsems[b])

        wait_gather(0)
        pltpu.sync_copy(rows_v.at[0], accum_sh.at[dst_v])

        def tail_body(g, carry):
            for b in range(NB):
                t = NB * g + b + 1
                bt = (b + 1) % NB
                blag = (b - 1) % NB

                @pl.when(t < S)
                def _step():
                    wait_gather(bt)
                    pltpu.async_copy(
                        rows_v.at[bt], accum_sh.at[dst_v], ssems[bt], add=True
                    )

                    @pl.when(t >= 3)
                    def _drain():
                        wait_scatter(blag)

                    @pl.when((t >= 2) & (t + NB - 2 < S))
                    def _refill():
                        pltpu.async_copy(
                            table_hbm.at[idx_v.at[t + NB - 2]],
                            rows_v.at[blag], gsems[blag],
                        )
            return carry

        lax.fori_loop(0, (S - 1 + NB - 1) // NB, tail_body, 0)

        wait_scatter((S - 2) % NB)
        wait_scatter((S - 1) % NB)

        acc_v = rows_v.at[0]
        pltpu.sync_copy(accum_sh.at[pl.ds(lbase, BT)], acc_v)
        inv = jnp.full((L,), 1.0 / S, jnp.float32)

        def sbody(r, carry):
            for j in range(D // L):
                acc_v[r, pl.ds(j * L, L)] = acc_v[r, pl.ds(j * L, L)] * inv
            return carry

        lax.fori_loop(0, BT, sbody, 0)
        pltpu.sync_copy(acc_v, out_hbm.at[pl.ds(gbase, BT)])

    return k(sentence, table)


def kernel(sentence, table):
    return _mean_embed(sentence, table)
